# baseline (speedup 1.0000x reference)
import functools

import jax
import jax.numpy as jnp
from jax import lax
from jax.experimental import pallas as pl
from jax.experimental.pallas import tpu as pltpu
from jax.experimental.pallas import tpu_sc as plsc

B = 16384
D = 32
NC = 2
NS = 16
NW = NC * NS
BPW = B // NW
CHUNK = 128
NCHUNK = BPW // CHUNK

@functools.lru_cache(maxsize=None)
def _make_sc_gather():
    mesh = plsc.VectorSubcoreMesh(
        core_axis_name="c", subcore_axis_name="s", num_cores=NC, num_subcores=NS
    )

    @functools.partial(
        pl.kernel,
        out_type=[jax.ShapeDtypeStruct((B, D), jnp.float32) for _ in range(4)],
        mesh=mesh,
        scratch_types=[
            pltpu.VMEM((NCHUNK, CHUNK), jnp.int32),
            pltpu.VMEM((NCHUNK, CHUNK), jnp.int32),
            pltpu.VMEM((BPW, D), jnp.float32),
            pltpu.VMEM((BPW, D), jnp.float32),
            pltpu.VMEM((BPW, D), jnp.float32),
            pltpu.VMEM((BPW, D), jnp.float32),
            pltpu.SemaphoreType.DMA,
        ],
        compiler_params=pltpu.CompilerParams(use_tc_tiling_on_sc=False),
    )
    def _sc_gather(uid_h, jid_h, umt_h, jmt_h, ugt_h, jgt_h,
                   out_um, out_jm, out_ug, out_jg,
                   uidx, jidx, bum, bjm, bug, bjg, sem):
        wid = lax.axis_index("s") * NC + lax.axis_index("c")
        r0 = wid * NCHUNK
        pltpu.sync_copy(uid_h.at[pl.ds(r0, NCHUNK)], uidx)
        pltpu.sync_copy(jid_h.at[pl.ds(r0, NCHUNK)], jidx)
        copies = []
        for c in range(NCHUNK):
            dst = pl.ds(c * CHUNK, CHUNK)
            copies.append(pltpu.async_copy(umt_h.at[uidx.at[c]], bum.at[dst], sem))
            copies.append(pltpu.async_copy(jmt_h.at[jidx.at[c]], bjm.at[dst], sem))
            copies.append(pltpu.async_copy(ugt_h.at[uidx.at[c]], bug.at[dst], sem))
            copies.append(pltpu.async_copy(jgt_h.at[jidx.at[c]], bjg.at[dst], sem))
        for cp in copies:
            cp.wait()
        base = wid * BPW
        pltpu.sync_copy(bum, out_um.at[pl.ds(base, BPW)])
        pltpu.sync_copy(bjm, out_jm.at[pl.ds(base, BPW)])
        pltpu.sync_copy(bug, out_ug.at[pl.ds(base, BPW)])
        pltpu.sync_copy(bjg, out_jg.at[pl.ds(base, BPW)])

    return _sc_gather


BLK = 2048


def _tc_body(um, jm, ug, jg, w1u, w1j, b1, w2, b2, w3, b3, w4, scal, out):
    x = jnp.maximum(um[:] @ w1u[:] + jm[:] @ w1j[:] + b1[:], 0.0)
    x = jnp.maximum(x @ w2[:] + b2[:], 0.0)
    x = jnp.maximum(x @ w3[:] + b3[:], 0.0)
    x = jnp.sum(x * w4[:], axis=1, keepdims=True) + scal[0]
    x = jnp.maximum(x, 0.0)
    u = ug[:]
    j = jg[:]
    dot = jnp.sum(u * j, axis=1, keepdims=True)
    su = jnp.sum(u * u, axis=1, keepdims=True)
    sj = jnp.sum(j * j, axis=1, keepdims=True)
    gmf = dot * lax.rsqrt(jnp.maximum(su, 1e-12)) * lax.rsqrt(jnp.maximum(sj, 1e-12))
    out[:] = x * scal[1] + gmf * scal[2] + scal[3]


def _tc_dense(um, jm, ug, jg, w1u, w1j, b1, w2, b2, w3, b3, w4, scal):
    row = pl.BlockSpec((BLK, D), lambda i: (i, 0))
    full = lambda a: pl.BlockSpec(a.shape, lambda i, _n=a.ndim: (0,) * _n)
    return pl.pallas_call(
        _tc_body,
        grid=(B // BLK,),
        in_specs=[row, row, row, row,
                  full(w1u), full(w1j), full(b1), full(w2), full(b2),
                  full(w3), full(b3), full(w4),
                  pl.BlockSpec(memory_space=pltpu.SMEM)],
        out_specs=pl.BlockSpec((BLK, 1), lambda i: (i, 0)),
        out_shape=jax.ShapeDtypeStruct((B, 1), jnp.float32),
    )(um, jm, ug, jg, w1u, w1j, b1, w2, b2, w3, b3, w4, scal)


def kernel(user_ids, joke_ids, user_mlp_table, joke_mlp_table,
           user_gmf_table, joke_gmf_table,
           W1, b1, W2, b2, W3, b3, W4, b4, W5, b5):
    uid = user_ids.astype(jnp.int32).reshape(B // CHUNK, CHUNK)
    jid = joke_ids.astype(jnp.int32).reshape(B // CHUNK, CHUNK)
    um, jm, ug, jg = _make_sc_gather()(uid, jid, user_mlp_table, joke_mlp_table,
                                       user_gmf_table, joke_gmf_table)
    w1u = W1[:D, :]
    w1j = W1[D:, :]
    scal = jnp.stack([b4[0], W5[0, 0], W5[1, 0], b5[0]])
    return _tc_dense(um, jm, ug, jg, w1u, w1j, b1.reshape(1, -1),
                     W2, b2.reshape(1, -1), W3, b3.reshape(1, -1),
                     W4.reshape(1, -1), scal)
